# baseline (speedup 1.0000x reference)
import functools

import jax
import jax.numpy as jnp
from jax import lax
from jax.experimental import pallas as pl
from jax.experimental.pallas import tpu as pltpu
from jax.experimental.pallas import tpu_sc as plsc

_B, _S, _H, _I = 1, 2048, 1024, 2048
_E, _K = 8, 2
_T = 256
_NB = 24
_P = _NB * _T
_NW = 32
_TPW = _S // _NW
_WL = 128


def _router_kernel(x_ref, gw_ref, logits_ref, wr1_ref, wr2_ref, p1_ref,
                   p2_ref, be_ref, nbt_ref):
    x = x_ref[...]
    logits = jnp.dot(x, gw_ref[...], preferred_element_type=jnp.float32)
    logits_ref[...] = logits
    col = lax.broadcasted_iota(jnp.int32, (_S, _E), 1)
    m1 = jnp.max(logits, axis=1, keepdims=True)
    a1 = jnp.min(jnp.where(logits == m1, col, _E), axis=1, keepdims=True)
    masked = jnp.where(col == a1, -jnp.inf, logits)
    m2 = jnp.max(masked, axis=1, keepdims=True)
    a2 = jnp.min(jnp.where(masked == m2, col, _E), axis=1, keepdims=True)
    e2 = jnp.exp(m2 - m1)
    wr1_ref[...] = jnp.broadcast_to(1.0 / (1.0 + e2), (_S, _WL))
    wr2_ref[...] = jnp.broadcast_to(e2 / (1.0 + e2), (_S, _WL))
    sel1 = col == a1
    sel2 = col == a2

    cc = jnp.where(sel1 | sel2, 1.0, 0.0)
    sh = 1
    while sh < _S:
        z = jnp.zeros((sh, _E), jnp.float32)
        cc = cc + jnp.concatenate([z, cc[:-sh, :]], axis=0)
        sh *= 2
    counts = cc[_S - 1:_S, :]
    nb = jnp.floor((counts + (_T - 1)) * (1.0 / _T))
    pend = nb
    sh = 1
    while sh < _E:
        z = jnp.zeros((1, sh), jnp.float32)
        pend = pend + jnp.concatenate([z, pend[:, :-sh]], axis=1)
        sh *= 2
    pstart = pend - nb
    nbt_ref[...] = pend[:, _E - 1:_E].astype(jnp.int32)
    rowb = lax.broadcasted_iota(jnp.int32, (_NB, _E), 0).astype(jnp.float32)
    be = jnp.sum(jnp.where(pend <= rowb, 1.0, 0.0), axis=1, keepdims=True)
    be_ref[...] = jnp.minimum(be, _E - 1.0).astype(jnp.int32)
    pos = pstart * _T + cc - 1.0
    p1_ref[...] = jnp.sum(jnp.where(sel1, pos, 0.0), axis=1,
                          keepdims=True).astype(jnp.int32)
    p2_ref[...] = jnp.sum(jnp.where(sel2, pos, 0.0), axis=1,
                          keepdims=True).astype(jnp.int32)


def _route(x32, gate_w):
    return pl.pallas_call(
        _router_kernel,
        out_shape=(
            jax.ShapeDtypeStruct((_S, _E), jnp.float32),
            jax.ShapeDtypeStruct((_S, _WL), jnp.float32),
            jax.ShapeDtypeStruct((_S, _WL), jnp.float32),
            jax.ShapeDtypeStruct((_S, 1), jnp.int32),
            jax.ShapeDtypeStruct((_S, 1), jnp.int32),
            jax.ShapeDtypeStruct((_NB, 1), jnp.int32),
            jax.ShapeDtypeStruct((1, 1), jnp.int32),
        ),
    )(x32, gate_w)


@functools.cache
def _sc_mesh():
    return plsc.VectorSubcoreMesh(core_axis_name="c", subcore_axis_name="s")


def _dispatch_body(x_hbm, wr1_hbm, wr2_hbm, p1_hbm, p2_hbm, xs_hbm, rw_hbm,
                   rows_v, wrow_v, idx_v, sem):
    wid = lax.axis_index("s") * 2 + lax.axis_index("c")
    base = wid * _TPW
    pltpu.sync_copy(x_hbm.at[pl.ds(base, _TPW)], rows_v)
    pltpu.sync_copy(p1_hbm.at[pl.ds(base, _TPW)], idx_v)
    pltpu.async_copy(rows_v, xs_hbm.at[idx_v], sem).wait()
    pltpu.sync_copy(wr1_hbm.at[pl.ds(base, _TPW)], wrow_v)
    pltpu.async_copy(wrow_v, rw_hbm.at[idx_v], sem).wait()
    pltpu.sync_copy(p2_hbm.at[pl.ds(base, _TPW)], idx_v)
    pltpu.async_copy(rows_v, xs_hbm.at[idx_v], sem).wait()
    pltpu.sync_copy(wr2_hbm.at[pl.ds(base, _TPW)], wrow_v)
    pltpu.async_copy(wrow_v, rw_hbm.at[idx_v], sem).wait()


def _dispatch(x32, wr1, wr2, p1, p2):
    return pl.kernel(
        _dispatch_body,
        out_type=(
            jax.ShapeDtypeStruct((_P, _H), jnp.float32),
            jax.ShapeDtypeStruct((_P, _WL), jnp.float32),
        ),
        mesh=_sc_mesh(),
        scratch_types=[
            pltpu.VMEM((_TPW, _H), jnp.float32),
            pltpu.VMEM((_TPW, _WL), jnp.float32),
            pltpu.VMEM((_TPW,), jnp.int32),
            pltpu.SemaphoreType.DMA,
        ],
    )(x32, wr1, wr2, p1, p2)


def _hid_kernel(be_ref, nbt_ref, x_ref, wg_ref, wu_ref, h_ref):
    b = pl.program_id(0)

    @pl.when(b < nbt_ref[0])
    def _():
        x = x_ref[...].astype(jnp.bfloat16)
        g = jnp.dot(x, wg_ref[0].astype(jnp.bfloat16),
                    preferred_element_type=jnp.float32)
        u = jnp.dot(x, wu_ref[0].astype(jnp.bfloat16),
                    preferred_element_type=jnp.float32)
        h_ref[...] = (jax.nn.gelu(g, approximate=True) * u).astype(jnp.bfloat16)


def _down_kernel(be_ref, nbt_ref, h_ref, rw_ref, wd_ref, y_ref):
    b = pl.program_id(0)

    @pl.when(b < nbt_ref[0])
    def _():
        y = jnp.dot(h_ref[...], wd_ref[0].astype(jnp.bfloat16),
                    preferred_element_type=jnp.float32)
        y_ref[...] = rw_ref[:, :1] * y


def _mlp(be, nbt, xs, rw, wg, wu, wd):
    h = pl.pallas_call(
        _hid_kernel,
        grid_spec=pltpu.PrefetchScalarGridSpec(
            num_scalar_prefetch=2,
            grid=(_NB,),
            in_specs=[
                pl.BlockSpec((_T, _H), lambda b, be, nbt: (b, 0)),
                pl.BlockSpec((1, _H, _I), lambda b, be, nbt: (be[b], 0, 0)),
                pl.BlockSpec((1, _H, _I), lambda b, be, nbt: (be[b], 0, 0)),
            ],
            out_specs=pl.BlockSpec((_T, _I), lambda b, be, nbt: (b, 0)),
        ),
        out_shape=jax.ShapeDtypeStruct((_P, _I), jnp.bfloat16),
        compiler_params=pltpu.CompilerParams(
            dimension_semantics=("arbitrary",),
        ),
    )(be, nbt, xs, wg, wu)
    return pl.pallas_call(
        _down_kernel,
        grid_spec=pltpu.PrefetchScalarGridSpec(
            num_scalar_prefetch=2,
            grid=(_NB,),
            in_specs=[
                pl.BlockSpec((_T, _I), lambda b, be, nbt: (b, 0)),
                pl.BlockSpec((_T, _WL), lambda b, be, nbt: (b, 0)),
                pl.BlockSpec((1, _I, _H), lambda b, be, nbt: (be[b], 0, 0)),
            ],
            out_specs=pl.BlockSpec((_T, _H), lambda b, be, nbt: (b, 0)),
        ),
        out_shape=jax.ShapeDtypeStruct((_P, _H), jnp.float32),
        compiler_params=pltpu.CompilerParams(
            dimension_semantics=("arbitrary",),
        ),
    )(be, nbt, h, rw, wd)


def _gather_body(y_hbm, p1_hbm, p2_hbm, y1_hbm, y2_hbm, r1_v,
                 i1_v, i2_v, sem1, sem2):
    wid = lax.axis_index("s") * 2 + lax.axis_index("c")
    base = wid * _TPW
    pltpu.sync_copy(p1_hbm.at[pl.ds(base, _TPW)], i1_v)
    pltpu.sync_copy(p2_hbm.at[pl.ds(base, _TPW)], i2_v)
    pltpu.async_copy(y_hbm.at[i1_v], r1_v, sem1).wait()
    pltpu.sync_copy(r1_v, y1_hbm.at[pl.ds(base, _TPW)])
    pltpu.async_copy(y_hbm.at[i2_v], r1_v, sem2).wait()
    pltpu.sync_copy(r1_v, y2_hbm.at[pl.ds(base, _TPW)])


def _gather(y, p1, p2):
    return pl.kernel(
        _gather_body,
        out_type=(
            jax.ShapeDtypeStruct((_S, _H), jnp.float32),
            jax.ShapeDtypeStruct((_S, _H), jnp.float32),
        ),
        mesh=_sc_mesh(),
        scratch_types=[
            pltpu.VMEM((_TPW, _H), jnp.float32),
            pltpu.VMEM((_TPW,), jnp.int32),
            pltpu.VMEM((_TPW,), jnp.int32),
            pltpu.SemaphoreType.DMA,
            pltpu.SemaphoreType.DMA,
        ],
    )(y, p1, p2)


def _combine_kernel(y1_ref, y2_ref, o_ref):
    o_ref[...] = y1_ref[...] + y2_ref[...]


def _combine(y1, y2):
    ts = 1024
    return pl.pallas_call(
        _combine_kernel,
        grid=(_S // ts,),
        in_specs=[
            pl.BlockSpec((ts, _H), lambda i: (i, 0)),
            pl.BlockSpec((ts, _H), lambda i: (i, 0)),
        ],
        out_specs=pl.BlockSpec((ts, _H), lambda i: (i, 0)),
        out_shape=jax.ShapeDtypeStruct((_S, _H), jnp.float32),
    )(y1, y2)


def kernel(hidden_states, gate_w, gate_proj_w, up_proj_w, down_proj_w):
    x32 = hidden_states.reshape(_S, _H)
    logits, wr1, wr2, p1, p2, be, nbt = _route(x32, gate_w)
    p1f = p1.reshape(_S)
    p2f = p2.reshape(_S)
    xs, rw = _dispatch(x32, wr1, wr2, p1f, p2f)
    y = _mlp(be.reshape(_NB), nbt.reshape(1), xs, rw,
             gate_proj_w, up_proj_w, down_proj_w)
    y1, y2 = _gather(y, p1f, p2f)
    out = _combine(y1, y2)
    return out.reshape(_B, _S, _H), logits.reshape(_B, _S, _E)


# speedup vs baseline: 1.0038x; 1.0038x over previous
import functools

import jax
import jax.numpy as jnp
from jax import lax
from jax.experimental import pallas as pl
from jax.experimental.pallas import tpu as pltpu
from jax.experimental.pallas import tpu_sc as plsc

_B, _S, _H, _I = 1, 2048, 1024, 2048
_E, _K = 8, 2
_T = 256
_NB = 24
_P = _NB * _T
_NW = 32
_TPW = _S // _NW
_WL = 128


def _router_kernel(x_ref, gw_ref, logits_ref, wr1_ref, wr2_ref, p1_ref,
                   p2_ref, be_ref, nbt_ref):
    x = x_ref[...]
    logits = jnp.dot(x, gw_ref[...], preferred_element_type=jnp.float32)
    logits_ref[...] = logits
    col = lax.broadcasted_iota(jnp.int32, (_S, _E), 1)
    m1 = jnp.max(logits, axis=1, keepdims=True)
    a1 = jnp.min(jnp.where(logits == m1, col, _E), axis=1, keepdims=True)
    masked = jnp.where(col == a1, -jnp.inf, logits)
    m2 = jnp.max(masked, axis=1, keepdims=True)
    a2 = jnp.min(jnp.where(masked == m2, col, _E), axis=1, keepdims=True)
    e2 = jnp.exp(m2 - m1)
    wr1_ref[...] = jnp.broadcast_to(1.0 / (1.0 + e2), (_S, _WL))
    wr2_ref[...] = jnp.broadcast_to(e2 / (1.0 + e2), (_S, _WL))
    sel1 = col == a1
    sel2 = col == a2

    cc = jnp.where(sel1 | sel2, 1.0, 0.0)
    sh = 1
    while sh < _S:
        z = jnp.zeros((sh, _E), jnp.float32)
        cc = cc + jnp.concatenate([z, cc[:-sh, :]], axis=0)
        sh *= 2
    counts = cc[_S - 1:_S, :]
    nb = jnp.floor((counts + (_T - 1)) * (1.0 / _T))
    pend = nb
    sh = 1
    while sh < _E:
        z = jnp.zeros((1, sh), jnp.float32)
        pend = pend + jnp.concatenate([z, pend[:, :-sh]], axis=1)
        sh *= 2
    pstart = pend - nb
    nbt_ref[...] = pend[:, _E - 1:_E].astype(jnp.int32)
    rowb = lax.broadcasted_iota(jnp.int32, (_NB, _E), 0).astype(jnp.float32)
    be = jnp.sum(jnp.where(pend <= rowb, 1.0, 0.0), axis=1, keepdims=True)
    be_ref[...] = jnp.minimum(be, _E - 1.0).astype(jnp.int32)
    pos = pstart * _T + cc - 1.0
    p1_ref[...] = jnp.sum(jnp.where(sel1, pos, 0.0), axis=1,
                          keepdims=True).astype(jnp.int32)
    p2_ref[...] = jnp.sum(jnp.where(sel2, pos, 0.0), axis=1,
                          keepdims=True).astype(jnp.int32)


def _route(x32, gate_w):
    return pl.pallas_call(
        _router_kernel,
        out_shape=(
            jax.ShapeDtypeStruct((_S, _E), jnp.float32),
            jax.ShapeDtypeStruct((_S, _WL), jnp.float32),
            jax.ShapeDtypeStruct((_S, _WL), jnp.float32),
            jax.ShapeDtypeStruct((_S, 1), jnp.int32),
            jax.ShapeDtypeStruct((_S, 1), jnp.int32),
            jax.ShapeDtypeStruct((_NB, 1), jnp.int32),
            jax.ShapeDtypeStruct((1, 1), jnp.int32),
        ),
    )(x32, gate_w)


@functools.cache
def _sc_mesh():
    return plsc.VectorSubcoreMesh(core_axis_name="c", subcore_axis_name="s")


def _dispatch_body(x_hbm, wr1_hbm, wr2_hbm, p1_hbm, p2_hbm, xs_hbm, rw_hbm,
                   rows_v, wrow_v, idx_v, sem):
    wid = lax.axis_index("s") * 2 + lax.axis_index("c")
    base = wid * _TPW
    pltpu.sync_copy(x_hbm.at[pl.ds(base, _TPW)], rows_v)
    pltpu.sync_copy(p1_hbm.at[pl.ds(base, _TPW)], idx_v)
    pltpu.async_copy(rows_v, xs_hbm.at[idx_v], sem).wait()
    pltpu.sync_copy(wr1_hbm.at[pl.ds(base, _TPW)], wrow_v)
    pltpu.async_copy(wrow_v, rw_hbm.at[idx_v], sem).wait()
    pltpu.sync_copy(p2_hbm.at[pl.ds(base, _TPW)], idx_v)
    pltpu.async_copy(rows_v, xs_hbm.at[idx_v], sem).wait()
    pltpu.sync_copy(wr2_hbm.at[pl.ds(base, _TPW)], wrow_v)
    pltpu.async_copy(wrow_v, rw_hbm.at[idx_v], sem).wait()


def _dispatch(x32, wr1, wr2, p1, p2):
    return pl.kernel(
        _dispatch_body,
        out_type=(
            jax.ShapeDtypeStruct((_P, _H), jnp.float32),
            jax.ShapeDtypeStruct((_P, _WL), jnp.float32),
        ),
        mesh=_sc_mesh(),
        scratch_types=[
            pltpu.VMEM((_TPW, _H), jnp.float32),
            pltpu.VMEM((_TPW, _WL), jnp.float32),
            pltpu.VMEM((_TPW,), jnp.int32),
            pltpu.SemaphoreType.DMA,
        ],
    )(x32, wr1, wr2, p1, p2)


def _hid_kernel(be_ref, nbt_ref, x_ref, wg_ref, wu_ref, h_ref):
    b = pl.program_id(0)

    @pl.when(b < nbt_ref[0])
    def _():
        x = x_ref[...].astype(jnp.bfloat16)
        g = jnp.dot(x, wg_ref[0].astype(jnp.bfloat16),
                    preferred_element_type=jnp.float32)
        u = jnp.dot(x, wu_ref[0].astype(jnp.bfloat16),
                    preferred_element_type=jnp.float32)
        h_ref[...] = (jax.nn.gelu(g, approximate=True) * u).astype(jnp.bfloat16)


def _down_kernel(be_ref, nbt_ref, h_ref, rw_ref, wd_ref, y_ref):
    b = pl.program_id(0)

    @pl.when(b < nbt_ref[0])
    def _():
        y = jnp.dot(h_ref[...], wd_ref[0].astype(jnp.bfloat16),
                    preferred_element_type=jnp.float32)
        y_ref[...] = rw_ref[:, :1] * y


def _mlp(be, nbt, xs, rw, wg, wu, wd):
    h = pl.pallas_call(
        _hid_kernel,
        grid_spec=pltpu.PrefetchScalarGridSpec(
            num_scalar_prefetch=2,
            grid=(_NB,),
            in_specs=[
                pl.BlockSpec((_T, _H), lambda b, be, nbt: (b, 0)),
                pl.BlockSpec((1, _H, _I), lambda b, be, nbt: (be[b], 0, 0)),
                pl.BlockSpec((1, _H, _I), lambda b, be, nbt: (be[b], 0, 0)),
            ],
            out_specs=pl.BlockSpec((_T, _I), lambda b, be, nbt: (b, 0)),
        ),
        out_shape=jax.ShapeDtypeStruct((_P, _I), jnp.bfloat16),
        compiler_params=pltpu.CompilerParams(
            dimension_semantics=("arbitrary",),
        ),
    )(be, nbt, xs, wg, wu)
    return pl.pallas_call(
        _down_kernel,
        grid_spec=pltpu.PrefetchScalarGridSpec(
            num_scalar_prefetch=2,
            grid=(_NB,),
            in_specs=[
                pl.BlockSpec((_T, _I), lambda b, be, nbt: (b, 0)),
                pl.BlockSpec((_T, _WL), lambda b, be, nbt: (b, 0)),
                pl.BlockSpec((1, _I, _H), lambda b, be, nbt: (be[b], 0, 0)),
            ],
            out_specs=pl.BlockSpec((_T, _H), lambda b, be, nbt: (b, 0)),
        ),
        out_shape=jax.ShapeDtypeStruct((_P, _H), jnp.float32),
        compiler_params=pltpu.CompilerParams(
            dimension_semantics=("arbitrary",),
        ),
    )(be, nbt, h, rw, wd)


_HALF = _TPW // 2


def _gather_body(y_hbm, p1_hbm, p2_hbm, o_hbm, r1_v, r2_v,
                 i1_v, i2_v, sem1, sem2):
    wid = lax.axis_index("s") * 2 + lax.axis_index("c")
    base = wid * _TPW
    for c in range(2):
        lo = base + c * _HALF
        pltpu.sync_copy(p1_hbm.at[pl.ds(lo, _HALF)], i1_v)
        pltpu.sync_copy(p2_hbm.at[pl.ds(lo, _HALF)], i2_v)
        c1 = pltpu.async_copy(y_hbm.at[i1_v], r1_v, sem1)
        c2 = pltpu.async_copy(y_hbm.at[i2_v], r2_v, sem2)
        c1.wait()
        c2.wait()

        def _row_add(j, carry):
            for k in range(_H // 16):
                sl = pl.ds(k * 16, 16)
                r1_v[j, sl] = r1_v[j, sl] + r2_v[j, sl]
            return carry

        lax.fori_loop(0, _HALF, _row_add, 0)
        pltpu.sync_copy(r1_v, o_hbm.at[pl.ds(lo, _HALF)])


def _gather(y, p1, p2):
    return pl.kernel(
        _gather_body,
        out_type=jax.ShapeDtypeStruct((_S, _H), jnp.float32),
        mesh=_sc_mesh(),
        scratch_types=[
            pltpu.VMEM((_HALF, _H), jnp.float32),
            pltpu.VMEM((_HALF, _H), jnp.float32),
            pltpu.VMEM((_HALF,), jnp.int32),
            pltpu.VMEM((_HALF,), jnp.int32),
            pltpu.SemaphoreType.DMA,
            pltpu.SemaphoreType.DMA,
        ],
    )(y, p1, p2)


def kernel(hidden_states, gate_w, gate_proj_w, up_proj_w, down_proj_w):
    x32 = hidden_states.reshape(_S, _H)
    logits, wr1, wr2, p1, p2, be, nbt = _route(x32, gate_w)
    p1f = p1.reshape(_S)
    p2f = p2.reshape(_S)
    xs, rw = _dispatch(x32, wr1, wr2, p1f, p2f)
    y = _mlp(be.reshape(_NB), nbt.reshape(1), xs, rw,
             gate_proj_w, up_proj_w, down_proj_w)
    out = _gather(y, p1f, p2f)
    return out.reshape(_B, _S, _H), logits.reshape(_B, _S, _E)


# speedup vs baseline: 1.0156x; 1.0118x over previous
import functools

import jax
import jax.numpy as jnp
from jax import lax
from jax.experimental import pallas as pl
from jax.experimental.pallas import tpu as pltpu
from jax.experimental.pallas import tpu_sc as plsc

_B, _S, _H, _I = 1, 2048, 1024, 2048
_E, _K = 8, 2
_T = 192
_NB = 29
_P = _NB * _T
_NW = 32
_TPW = _S // _NW
_WL = 128


def _router_kernel(x_ref, gw_ref, logits_ref, wr1_ref, wr2_ref, p1_ref,
                   p2_ref, be_ref, nbt_ref):
    x = x_ref[...]
    logits = jnp.dot(x, gw_ref[...], preferred_element_type=jnp.float32)
    logits_ref[...] = logits
    col = lax.broadcasted_iota(jnp.int32, (_S, _E), 1)
    m1 = jnp.max(logits, axis=1, keepdims=True)
    a1 = jnp.min(jnp.where(logits == m1, col, _E), axis=1, keepdims=True)
    masked = jnp.where(col == a1, -jnp.inf, logits)
    m2 = jnp.max(masked, axis=1, keepdims=True)
    a2 = jnp.min(jnp.where(masked == m2, col, _E), axis=1, keepdims=True)
    e2 = jnp.exp(m2 - m1)
    wr1_ref[...] = jnp.broadcast_to(1.0 / (1.0 + e2), (_S, _WL))
    wr2_ref[...] = jnp.broadcast_to(e2 / (1.0 + e2), (_S, _WL))
    sel1 = col == a1
    sel2 = col == a2

    cc = jnp.where(sel1 | sel2, 1.0, 0.0)
    sh = 1
    while sh < _S:
        z = jnp.zeros((sh, _E), jnp.float32)
        cc = cc + jnp.concatenate([z, cc[:-sh, :]], axis=0)
        sh *= 2
    counts = cc[_S - 1:_S, :]
    nb = jnp.floor((counts + (_T - 1)) * (1.0 / _T))
    pend = nb
    sh = 1
    while sh < _E:
        z = jnp.zeros((1, sh), jnp.float32)
        pend = pend + jnp.concatenate([z, pend[:, :-sh]], axis=1)
        sh *= 2
    pstart = pend - nb
    nbt_ref[...] = pend[:, _E - 1:_E].astype(jnp.int32)
    rowb = lax.broadcasted_iota(jnp.int32, (_NB, _E), 0).astype(jnp.float32)
    be = jnp.sum(jnp.where(pend <= rowb, 1.0, 0.0), axis=1, keepdims=True)
    be_ref[...] = jnp.minimum(be, _E - 1.0).astype(jnp.int32)
    pos = pstart * _T + cc - 1.0
    p1_ref[...] = jnp.sum(jnp.where(sel1, pos, 0.0), axis=1,
                          keepdims=True).astype(jnp.int32)
    p2_ref[...] = jnp.sum(jnp.where(sel2, pos, 0.0), axis=1,
                          keepdims=True).astype(jnp.int32)


def _route(x32, gate_w):
    return pl.pallas_call(
        _router_kernel,
        out_shape=(
            jax.ShapeDtypeStruct((_S, _E), jnp.float32),
            jax.ShapeDtypeStruct((_S, _WL), jnp.float32),
            jax.ShapeDtypeStruct((_S, _WL), jnp.float32),
            jax.ShapeDtypeStruct((_S, 1), jnp.int32),
            jax.ShapeDtypeStruct((_S, 1), jnp.int32),
            jax.ShapeDtypeStruct((_NB, 1), jnp.int32),
            jax.ShapeDtypeStruct((1, 1), jnp.int32),
        ),
    )(x32, gate_w)


@functools.cache
def _sc_mesh():
    return plsc.VectorSubcoreMesh(core_axis_name="c", subcore_axis_name="s")


def _dispatch_body(x_hbm, wr1_hbm, wr2_hbm, p1_hbm, p2_hbm, xs_hbm, rw_hbm,
                   rows_v, w1row_v, w2row_v, i1_v, i2_v, sem1, sem2):
    wid = lax.axis_index("s") * 2 + lax.axis_index("c")
    base = wid * _TPW
    pltpu.sync_copy(x_hbm.at[pl.ds(base, _TPW)], rows_v)
    pltpu.sync_copy(p1_hbm.at[pl.ds(base, _TPW)], i1_v)
    pltpu.sync_copy(p2_hbm.at[pl.ds(base, _TPW)], i2_v)
    pltpu.sync_copy(wr1_hbm.at[pl.ds(base, _TPW)], w1row_v)
    pltpu.sync_copy(wr2_hbm.at[pl.ds(base, _TPW)], w2row_v)
    c1 = pltpu.async_copy(rows_v, xs_hbm.at[i1_v], sem1)
    c2 = pltpu.async_copy(rows_v, xs_hbm.at[i2_v], sem1)
    c3 = pltpu.async_copy(w1row_v, rw_hbm.at[i1_v], sem2)
    c4 = pltpu.async_copy(w2row_v, rw_hbm.at[i2_v], sem2)
    c1.wait()
    c2.wait()
    c3.wait()
    c4.wait()


def _dispatch(x32, wr1, wr2, p1, p2):
    return pl.kernel(
        _dispatch_body,
        out_type=(
            jax.ShapeDtypeStruct((_P, _H), jnp.float32),
            jax.ShapeDtypeStruct((_P, _WL), jnp.float32),
        ),
        mesh=_sc_mesh(),
        scratch_types=[
            pltpu.VMEM((_TPW, _H), jnp.float32),
            pltpu.VMEM((_TPW, _WL), jnp.float32),
            pltpu.VMEM((_TPW, _WL), jnp.float32),
            pltpu.VMEM((_TPW,), jnp.int32),
            pltpu.VMEM((_TPW,), jnp.int32),
            pltpu.SemaphoreType.DMA,
            pltpu.SemaphoreType.DMA,
        ],
    )(x32, wr1, wr2, p1, p2)


def _hid_kernel(be_ref, nbt_ref, x_ref, wg_ref, wu_ref, h_ref):
    b = pl.program_id(0)

    @pl.when(b < nbt_ref[0])
    def _():
        x = x_ref[...].astype(jnp.bfloat16)
        g = jnp.dot(x, wg_ref[0].astype(jnp.bfloat16),
                    preferred_element_type=jnp.float32)
        u = jnp.dot(x, wu_ref[0].astype(jnp.bfloat16),
                    preferred_element_type=jnp.float32)
        h_ref[...] = (jax.nn.gelu(g, approximate=True) * u).astype(jnp.bfloat16)


def _down_kernel(be_ref, nbt_ref, h_ref, rw_ref, wd_ref, y_ref):
    b = pl.program_id(0)

    @pl.when(b < nbt_ref[0])
    def _():
        y = jnp.dot(h_ref[...], wd_ref[0].astype(jnp.bfloat16),
                    preferred_element_type=jnp.float32)
        y_ref[...] = rw_ref[:, :1] * y


def _mlp(be, nbt, xs, rw, wg, wu, wd):
    h = pl.pallas_call(
        _hid_kernel,
        grid_spec=pltpu.PrefetchScalarGridSpec(
            num_scalar_prefetch=2,
            grid=(_NB,),
            in_specs=[
                pl.BlockSpec((_T, _H), lambda b, be, nbt: (b, 0)),
                pl.BlockSpec((1, _H, _I), lambda b, be, nbt: (be[b], 0, 0)),
                pl.BlockSpec((1, _H, _I), lambda b, be, nbt: (be[b], 0, 0)),
            ],
            out_specs=pl.BlockSpec((_T, _I), lambda b, be, nbt: (b, 0)),
        ),
        out_shape=jax.ShapeDtypeStruct((_P, _I), jnp.bfloat16),
        compiler_params=pltpu.CompilerParams(
            dimension_semantics=("arbitrary",),
        ),
    )(be, nbt, xs, wg, wu)
    return pl.pallas_call(
        _down_kernel,
        grid_spec=pltpu.PrefetchScalarGridSpec(
            num_scalar_prefetch=2,
            grid=(_NB,),
            in_specs=[
                pl.BlockSpec((_T, _I), lambda b, be, nbt: (b, 0)),
                pl.BlockSpec((_T, _WL), lambda b, be, nbt: (b, 0)),
                pl.BlockSpec((1, _I, _H), lambda b, be, nbt: (be[b], 0, 0)),
            ],
            out_specs=pl.BlockSpec((_T, _H), lambda b, be, nbt: (b, 0)),
        ),
        out_shape=jax.ShapeDtypeStruct((_P, _H), jnp.float32),
        compiler_params=pltpu.CompilerParams(
            dimension_semantics=("arbitrary",),
        ),
    )(be, nbt, h, rw, wd)


_HALF = _TPW // 2


def _gather_body(y_hbm, p1_hbm, p2_hbm, o_hbm, r1_v, r2_v,
                 i1_v, i2_v, sem1, sem2):
    wid = lax.axis_index("s") * 2 + lax.axis_index("c")
    base = wid * _TPW
    for c in range(2):
        lo = base + c * _HALF
        pltpu.sync_copy(p1_hbm.at[pl.ds(lo, _HALF)], i1_v)
        pltpu.sync_copy(p2_hbm.at[pl.ds(lo, _HALF)], i2_v)
        c1 = pltpu.async_copy(y_hbm.at[i1_v], r1_v, sem1)
        c2 = pltpu.async_copy(y_hbm.at[i2_v], r2_v, sem2)
        c1.wait()
        c2.wait()

        def _row_add(j, carry):
            for k in range(_H // 16):
                sl = pl.ds(k * 16, 16)
                r1_v[j, sl] = r1_v[j, sl] + r2_v[j, sl]
            return carry

        lax.fori_loop(0, _HALF, _row_add, 0)
        pltpu.sync_copy(r1_v, o_hbm.at[pl.ds(lo, _HALF)])


def _gather(y, p1, p2):
    return pl.kernel(
        _gather_body,
        out_type=jax.ShapeDtypeStruct((_S, _H), jnp.float32),
        mesh=_sc_mesh(),
        scratch_types=[
            pltpu.VMEM((_HALF, _H), jnp.float32),
            pltpu.VMEM((_HALF, _H), jnp.float32),
            pltpu.VMEM((_HALF,), jnp.int32),
            pltpu.VMEM((_HALF,), jnp.int32),
            pltpu.SemaphoreType.DMA,
            pltpu.SemaphoreType.DMA,
        ],
    )(y, p1, p2)


def kernel(hidden_states, gate_w, gate_proj_w, up_proj_w, down_proj_w):
    x32 = hidden_states.reshape(_S, _H)
    logits, wr1, wr2, p1, p2, be, nbt = _route(x32, gate_w)
    p1f = p1.reshape(_S)
    p2f = p2.reshape(_S)
    xs, rw = _dispatch(x32, wr1, wr2, p1f, p2f)
    y = _mlp(be.reshape(_NB), nbt.reshape(1), xs, rw,
             gate_proj_w, up_proj_w, down_proj_w)
    out = _gather(y, p1f, p2f)
    return out.reshape(_B, _S, _H), logits.reshape(_B, _S, _E)
